# Initial kernel scaffold; baseline (speedup 1.0000x reference)
import jax
import jax.numpy as jnp
from jax.experimental import pallas as pl


def kernel(queue):
    raise NotImplementedError("write your pallas kernel here")


# speedup vs baseline: 1.1150x; 1.1150x over previous
import jax
import jax.numpy as jnp
from jax.experimental import pallas as pl

_F = 128
_B = 2048


def _transpose_body(x_ref, o_ref):
    o_ref[...] = x_ref[...].T


def kernel(queue):
    f, k = queue.shape
    return pl.pallas_call(
        _transpose_body,
        grid=(k // _B,),
        in_specs=[pl.BlockSpec((f, _B), lambda i: (0, i))],
        out_specs=pl.BlockSpec((_B, f), lambda i: (i, 0)),
        out_shape=jax.ShapeDtypeStruct((k, f), queue.dtype),
    )(queue)


# speedup vs baseline: 1.5682x; 1.4064x over previous
import jax
import jax.numpy as jnp
from jax.experimental import pallas as pl

_F = 128
_B = 4096


def _transpose_body(x_ref, o_ref):
    o_ref[...] = x_ref[...].T


def kernel(queue):
    f, k = queue.shape
    return pl.pallas_call(
        _transpose_body,
        grid=(k // _B,),
        in_specs=[pl.BlockSpec((f, _B), lambda i: (0, i))],
        out_specs=pl.BlockSpec((_B, f), lambda i: (i, 0)),
        out_shape=jax.ShapeDtypeStruct((k, f), queue.dtype),
    )(queue)


# speedup vs baseline: 1.7776x; 1.1336x over previous
import jax
import jax.numpy as jnp
from jax.experimental import pallas as pl

_F = 128
_B = 8192


def _transpose_body(x_ref, o_ref):
    o_ref[...] = x_ref[...].T


def kernel(queue):
    f, k = queue.shape
    return pl.pallas_call(
        _transpose_body,
        grid=(k // _B,),
        in_specs=[pl.BlockSpec((f, _B), lambda i: (0, i))],
        out_specs=pl.BlockSpec((_B, f), lambda i: (i, 0)),
        out_shape=jax.ShapeDtypeStruct((k, f), queue.dtype),
    )(queue)


# speedup vs baseline: 1.8066x; 1.0163x over previous
import jax
import jax.numpy as jnp
from jax.experimental import pallas as pl

_F = 128
_B = 16384


def _transpose_body(x_ref, o_ref):
    o_ref[...] = x_ref[...].T


def kernel(queue):
    f, k = queue.shape
    return pl.pallas_call(
        _transpose_body,
        grid=(k // _B,),
        in_specs=[pl.BlockSpec((f, _B), lambda i: (0, i))],
        out_specs=pl.BlockSpec((_B, f), lambda i: (i, 0)),
        out_shape=jax.ShapeDtypeStruct((k, f), queue.dtype),
    )(queue)
